# Initial kernel scaffold; baseline (speedup 1.0000x reference)
import jax
import jax.numpy as jnp
from jax.experimental import pallas as pl


def kernel(guidance, ini_depth, sparse_depth, We1, be1, Wa1, ba1, We2, be2, Wa2, ba2, We3, be3, Wa3, ba3):
    raise NotImplementedError("write your pallas kernel here")


# speedup vs baseline: 12.2004x; 12.2004x over previous
import functools

import jax
import jax.numpy as jnp
import numpy as np
from jax import lax
from jax.experimental import pallas as pl
from jax.experimental.pallas import tpu as pltpu
from jax.experimental.pallas import tpu_sc as plsc

N_REAL = 7752
NP = 7808
NP2 = 8192
BM = 128
NBLK = NP // BM
K = 16

NW = 32
NWK = NP2 // NW
NBG = 128
NCHG = NWK // NBG


def _camera_np():
    xx, yy = np.meshgrid(np.arange(0, 102, 1), np.arange(0, 76, 1))
    fx_d = 582.6244816773795 / 2.0
    fy_d = 582.6910327098864 / 2.0
    cx_d = 313.0447587080473 / 2.0
    cy_d = 238.44389626620386 / 2.0
    x_3d = ((xx - cx_d) / fx_d).astype(np.float32)
    y_3d = ((yy - cy_d) / fy_d).astype(np.float32)
    return x_3d, y_3d


_X3D, _Y3D = _camera_np()


def _knn_body(f_ref, ft_ref, idx_ref, d_scr):
    fb = f_ref[...]
    ft = ft_ref[...]
    dd = lax.dot_general(fb, ft, (((1,), (0,)), ((), ())),
                         precision=lax.Precision.DEFAULT,
                         preferred_element_type=jnp.float32)
    sqi = jnp.sum(fb * fb, axis=1, keepdims=True)
    sqj = jnp.sum(ft * ft, axis=0, keepdims=True)
    d = (sqi + (-2.0) * dd) + sqj
    colm = lax.broadcasted_iota(jnp.int32, (BM, NP), 1)
    d = jnp.where(colm >= N_REAL, jnp.float32(jnp.inf), d)
    d_scr[...] = d

    picks = []
    for _t in range(K):
        dcur = d_scr[...]
        m = jnp.min(dcur, axis=1, keepdims=True)
        cand = jnp.where(dcur == m, colm, jnp.int32(NP))
        amin = jnp.min(cand, axis=1, keepdims=True)
        picks.append(amin)
        d_scr[...] = jnp.where(colm == amin, jnp.float32(jnp.inf), dcur)
    idx_ref[...] = jnp.concatenate(picks, axis=1)


def _knn(F, FT):
    Cf = F.shape[1]
    return pl.pallas_call(
        _knn_body,
        grid=(NBLK,),
        in_specs=[
            pl.BlockSpec((BM, Cf), lambda i: (i, 0)),
            pl.BlockSpec((Cf, NP), lambda i: (0, 0)),
        ],
        out_specs=pl.BlockSpec((BM, K), lambda i: (i, 0)),
        out_shape=jax.ShapeDtypeStruct((NP, K), jnp.int32),
        scratch_shapes=[pltpu.VMEM((BM, NP), jnp.float32)],
    )(F, FT)


def _sc_gather(idxT, a_tab):
    mesh = plsc.VectorSubcoreMesh(core_axis_name="c", subcore_axis_name="s")

    @functools.partial(
        pl.kernel,
        out_type=jax.ShapeDtypeStruct((K, NP2, 128), jnp.float32),
        mesh=mesh,
        scratch_types=[
            pltpu.VMEM((K, NBG), jnp.int32),
            pltpu.VMEM((2, NBG, 128), jnp.float32),
            pltpu.SemaphoreType.DMA,
        ],
    )
    def sck(idx_hbm, a_hbm, xj_hbm, idx_v, rows_v, sem):
        wid = lax.axis_index("s") * 2 + lax.axis_index("c")
        base = wid * NWK

        def chunk(ci, carry):
            nb0 = base + ci * NBG
            pltpu.sync_copy(idx_hbm.at[:, pl.ds(nb0, NBG)], idx_v)
            d = pltpu.async_copy(a_hbm.at[idx_v.at[0]], rows_v.at[0], sem)
            for k in range(K):
                if k + 1 < K:
                    d_next = pltpu.async_copy(
                        a_hbm.at[idx_v.at[k + 1]], rows_v.at[(k + 1) % 2], sem)
                d.wait()
                pltpu.sync_copy(rows_v.at[k % 2], xj_hbm.at[k, pl.ds(nb0, NBG)])
                if k + 1 < K:
                    d = d_next
            return carry

        lax.fori_loop(0, NCHG, chunk, 0)

    return sck(idxT, a_tab)


def _edge_attn_body(a_ref, xj_ref, w_ref, b_ref, h_ref):
    xi = a_ref[...]
    w = w_ref[...]
    b = b_ref[...]
    ems = []
    for k in range(K):
        xj = xj_ref[k]
        cat = jnp.concatenate([xi, xj - xi], axis=1)
        em = lax.dot_general(cat, w, (((1,), (0,)), ((), ())),
                             precision=lax.Precision.DEFAULT,
                             preferred_element_type=jnp.float32) + b
        ems.append(em)
    atts = [em[:, 128:] for em in ems]
    m = atts[0]
    for k in range(1, K):
        m = jnp.maximum(m, atts[k])
    ws = [jnp.exp(a - m) for a in atts]
    s = ws[0]
    for k in range(1, K):
        s = s + ws[k]
    out = ems[0][:, :128] * (ws[0] / s)
    for k in range(1, K):
        out = out + ems[k][:, :128] * (ws[k] / s)
    h_ref[...] = out


def _edge_attn(A, XJ, Wcat, bcat):
    return pl.pallas_call(
        _edge_attn_body,
        grid=(NBLK,),
        in_specs=[
            pl.BlockSpec((BM, 128), lambda i: (i, 0)),
            pl.BlockSpec((K, BM, 128), lambda i: (0, i, 0)),
            pl.BlockSpec((256, 256), lambda i: (0, 0)),
            pl.BlockSpec((1, 256), lambda i: (0, 0)),
        ],
        out_specs=pl.BlockSpec((BM, 128), lambda i: (i, 0)),
        out_shape=jax.ShapeDtypeStruct((NP, 128), jnp.float32),
    )(A, XJ, Wcat, bcat)


def _make_layer_weights(We, Wa, be, ba, cin):
    co = We.shape[0]
    Wcat = jnp.zeros((256, 256), jnp.float32)
    Wcat = Wcat.at[:cin, :co].set(We[:, :cin].T)
    Wcat = Wcat.at[:cin, 128:128 + co].set(Wa[:, :cin].T)
    Wcat = Wcat.at[128:128 + cin, :co].set(We[:, cin:].T)
    Wcat = Wcat.at[128:128 + cin, 128:128 + co].set(Wa[:, cin:].T)
    bcat = jnp.concatenate([jnp.pad(be, (0, 128 - co)),
                            jnp.pad(ba, (0, 128 - co))])[None, :]
    return Wcat, bcat


def _layer(F, A, Wcat, bcat):
    idx = _knn(F, jnp.transpose(F))
    idxT = jnp.transpose(jnp.pad(idx, ((0, NP2 - NP), (0, 0))))
    XJ = _sc_gather(idxT, A)
    return _edge_attn(A, XJ, Wcat, bcat)


def kernel(guidance, ini_depth, sparse_depth, We1, be1, Wa1, ba1,
           We2, be2, Wa2, ba2, We3, be3, Wa3, ba3):
    def _r(v):
        return v.astype(jnp.bfloat16).astype(jnp.float32)

    mask = jnp.sign(sparse_depth)
    ini = (1.0 - mask) * ini_depth + mask * sparse_depth
    ini_s = _r(ini[0, 0, 1::3, 0::3])
    x3 = jnp.asarray(_X3D) * ini_s * 3.0
    y3 = jnp.asarray(_Y3D) * ini_s * 3.0
    loc = jnp.stack([x3, y3, ini_s], axis=-1).reshape(N_REAL, 3)

    gp = jnp.pad(guidance[0], ((0, 0), (3, 3), (4, 4)))
    g = jnp.stack(
        [gp[t, t // 9:t // 9 + 226:3, t % 9:t % 9 + 304:3] for t in range(81)],
        axis=-1)
    A1 = _r(g.reshape(N_REAL, 81))

    F1 = jnp.pad(loc, ((0, NP - N_REAL), (0, 5)))
    A1p = jnp.pad(A1, ((0, NP - N_REAL), (0, 47)))

    Wc1, bc1 = _make_layer_weights(We1, Wa1, be1, ba1, 81)
    Wc2, bc2 = _make_layer_weights(We2, Wa2, be2, ba2, 96)
    Wc3, bc3 = _make_layer_weights(We3, Wa3, be3, ba3, 96)

    h1 = _layer(F1, A1p, Wc1, bc1)
    h2 = _layer(h1, h1, Wc2, bc2)
    h3 = _layer(h2, h2, Wc3, bc3)

    hout = h3[:N_REAL, :9].reshape(76, 102, 3, 3)
    img = _r(hout.transpose(0, 2, 1, 3).reshape(228, 306))
    return img[:, 1:305][None, None]
